# baseline (speedup 1.0000x reference)
import dataclasses
import functools

import jax
import jax.numpy as jnp
from jax import lax
from jax.experimental import pallas as pl
from jax.experimental.pallas import tpu as pltpu
from jax.experimental.pallas import tpu_sc as plsc

B = 16384
F = 26
V = 100000
D = 16
BF = B * F
ROWS = BF // 128
NW = 32
ROWS_PER_W = ROWS // NW
CHUNK_R = 8
STEPS = ROWS_PER_W // CHUNK_R

_i32 = jnp.int32
_f32 = jnp.float32


def _sc_gather(xi_flat, t2_flat, t1_flat):
    mesh = plsc.VectorSubcoreMesh(core_axis_name="c", subcore_axis_name="s")
    cp = pltpu.CompilerParams()
    if "needs_layout_passes" in pltpu.CompilerParams.__dataclass_fields__:
        cp = dataclasses.replace(cp, needs_layout_passes=False)
    if "use_tc_tiling_on_sc" in pltpu.CompilerParams.__dataclass_fields__:
        cp = dataclasses.replace(cp, use_tc_tiling_on_sc=False)

    @functools.partial(
        pl.kernel,
        compiler_params=cp,
        out_type=(
            jax.ShapeDtypeStruct((ROWS, 128, D), _f32),
            jax.ShapeDtypeStruct((ROWS, 128), _f32),
        ),
        mesh=mesh,
        scratch_types=[
            pltpu.VMEM((CHUNK_R, 128), _i32),
            pltpu.VMEM((CHUNK_R, 128), _i32),
            pltpu.VMEM((CHUNK_R, 128), _i32),
            pltpu.VMEM((CHUNK_R, 128), _i32),
            pltpu.VMEM((CHUNK_R, 128, D), _f32),
            pltpu.VMEM((CHUNK_R, 128, D), _f32),
            pltpu.VMEM((CHUNK_R, 128), _f32),
            pltpu.SemaphoreType.DMA,
        ],
    )
    def sc_kernel(xi_hbm, t2_hbm, t1_hbm, g2_hbm, g1_hbm,
                  idx_v, sidx_v, fidx_v, lane_v, rows_v, frows_v, g1_v, sem):
        wid = lax.axis_index("c") * 16 + lax.axis_index("s")
        iota = lax.broadcasted_iota(_i32, (16,), 0)

        @pl.loop(0, STEPS)
        def _step(c):
            base_row = wid * ROWS_PER_W + c * CHUNK_R
            pltpu.sync_copy(xi_hbm.at[pl.ds(base_row, CHUNK_R)], idx_v)

            @pl.loop(0, CHUNK_R)
            def _prep(j):
                for l in range(8):
                    sl = (j, pl.ds(l * 16, 16))
                    pos = (base_row + j) * 128 + l * 16 + iota
                    s = (pos % F) * V + idx_v[sl]
                    sidx_v[sl] = s
                    fidx_v[sl] = lax.shift_right_logical(s, 4)
                    lane_v[sl] = lax.bitwise_and(s, 15)

            @pl.loop(0, CHUNK_R)
            def _gather(j):
                d2 = pltpu.async_copy(t2_hbm.at[sidx_v.at[j]], rows_v.at[j], sem)
                d1 = pltpu.async_copy(t1_hbm.at[fidx_v.at[j]], frows_v.at[j], sem)
                d2.wait()
                d1.wait()

            @pl.loop(0, CHUNK_R)
            def _select(j):
                jv = jnp.full((16,), j, dtype=_i32)
                for l in range(8):
                    sl = (j, pl.ds(l * 16, 16))
                    g1_v[sl] = plsc.load_gather(
                        frows_v, [jv, l * 16 + iota, lane_v[sl]])

            pltpu.sync_copy(rows_v, g2_hbm.at[pl.ds(base_row, CHUNK_R)])
            pltpu.sync_copy(g1_v, g1_hbm.at[pl.ds(base_row, CHUNK_R)])

    return sc_kernel(xi_flat, t2_flat, t1_flat)


BB = 1024


def _tc_body(xv_ref, g1_ref, g2_ref, w1_ref, b1_ref, w2_ref, b2_ref,
             bias_ref, out_ref):
    xv = xv_ref[...]
    g1 = g1_ref[...]
    g2 = g2_ref[...]
    hi = jax.lax.Precision.HIGHEST

    ef = lax.broadcasted_iota(_i32, (F, F * D), 0)
    ej = lax.broadcasted_iota(_i32, (F, F * D), 1)
    E = (lax.shift_right_logical(ej, 4) == ef).astype(_f32)
    xe = jnp.dot(xv, E, precision=hi, preferred_element_type=_f32)
    fm2 = g2 * xe

    sj = lax.broadcasted_iota(_i32, (F * D, D), 0)
    sd = lax.broadcasted_iota(_i32, (F * D, D), 1)
    S = (lax.bitwise_and(sj, 15) == sd).astype(_f32)
    fm_sum = jnp.dot(fm2, S, precision=hi, preferred_element_type=_f32)
    fm_sq = jnp.dot(fm2 * fm2, S, precision=hi, preferred_element_type=_f32)
    second = 0.5 * jnp.sum(fm_sum * fm_sum - fm_sq, axis=1)

    z1 = jnp.maximum(
        jnp.dot(fm2, w1_ref[...], precision=hi, preferred_element_type=_f32)
        + b1_ref[...], 0.0)
    z2 = jnp.maximum(
        jnp.dot(z1, w2_ref[...], precision=hi, preferred_element_type=_f32)
        + b2_ref[...], 0.0)
    deep = jnp.sum(z2, axis=1)

    first = jnp.sum(g1 * xv, axis=1)
    out_ref[0, 0, :] = first + second + deep + bias_ref[0, 0]


def kernel(Xi, Xv, first_tables, second_tables, W1, b1, W2, b2, bias):
    xi_flat = Xi.reshape(BF).astype(_i32).reshape(ROWS, 128)
    t2_flat = second_tables.reshape(F * V, D)
    t1_flat = first_tables.reshape(F * V // D, D)

    g2, g1 = _sc_gather(xi_flat, t2_flat, t1_flat)
    g2r = g2.reshape(B, F * D)
    g1r = g1.reshape(B, F)

    grid = (B // BB,)
    out = pl.pallas_call(
        _tc_body,
        grid=grid,
        in_specs=[
            pl.BlockSpec((BB, F), lambda i: (i, 0)),
            pl.BlockSpec((BB, F), lambda i: (i, 0)),
            pl.BlockSpec((BB, F * D), lambda i: (i, 0)),
            pl.BlockSpec((F * D, 32), lambda i: (0, 0)),
            pl.BlockSpec((1, 32), lambda i: (0, 0)),
            pl.BlockSpec((32, 32), lambda i: (0, 0)),
            pl.BlockSpec((1, 32), lambda i: (0, 0)),
            pl.BlockSpec((1, 1), lambda i: (0, 0)),
        ],
        out_specs=pl.BlockSpec((1, 1, BB), lambda i: (i, 0, 0)),
        out_shape=jax.ShapeDtypeStruct((B // BB, 1, BB), _f32),
        compiler_params=pltpu.CompilerParams(
            dimension_semantics=("arbitrary",)),
    )(Xv, g1r, g2r, W1, b1.reshape(1, 32), W2, b2.reshape(1, 32),
      bias.reshape(1, 1))
    return out.reshape(B)
